# baseline (speedup 1.0000x reference)
import jax
import jax.numpy as jnp
from jax import lax
from jax.experimental import pallas as pl
from jax.experimental.pallas import tpu as pltpu
from jax.experimental.pallas import tpu_sc as plsc

N = 10000
K = 32
D = 128
OUT = 128

S = 9200
BN = 400
BT = 800
G = 8
NW = 32
NC = 2


def _fused_body(mb_ref, nf_ref, w_ref, b_ref, out_ref):
    v = mb_ref[...]
    w = jnp.maximum(
        jnp.maximum(v[:, 0:8, :], v[:, 8:16, :]),
        jnp.maximum(v[:, 16:24, :], v[:, 24:32, :]),
    )
    acc = jnp.max(w, axis=1)
    dn = (((1,), (1,)), ((), ()))
    out = lax.dot_general(acc, w_ref[:, :D], dn, preferred_element_type=jnp.float32)
    out += lax.dot_general(nf_ref[...], w_ref[:, D:], dn, preferred_element_type=jnp.float32)
    out_ref[...] = out + b_ref[...]


def _tc_fused(mailbox_h, node_feat, W, b2, rows):
    grid = rows // BN
    return pl.pallas_call(
        _fused_body,
        grid=(grid,),
        in_specs=[
            pl.BlockSpec((BN, K, D), lambda i: (i, 0, 0)),
            pl.BlockSpec((BN, D), lambda i: (i, 0)),
            pl.BlockSpec((OUT, 2 * D), lambda i: (0, 0)),
            pl.BlockSpec((1, OUT), lambda i: (0, 0)),
        ],
        out_specs=pl.BlockSpec((BN, OUT), lambda i: (i, 0)),
        out_shape=jax.ShapeDtypeStruct((N, OUT), jnp.float32),
    )(mailbox_h, node_feat, W, b2)


def _linear_body(full_ref, h_ref, nf_ref, w_ref, b_ref, out_ref):
    del full_ref
    dn = (((1,), (1,)), ((), ()))
    out = lax.dot_general(h_ref[...], w_ref[:, :D], dn, preferred_element_type=jnp.float32)
    out += lax.dot_general(nf_ref[...], w_ref[:, D:], dn, preferred_element_type=jnp.float32)
    out_ref[...] = out + b_ref[...]


def _tc_linear(head_out, h_pool, node_feat, W, b2, rows):
    grid = rows // BT
    off = S // BT
    return pl.pallas_call(
        _linear_body,
        grid=(grid,),
        in_specs=[
            pl.BlockSpec(memory_space=pl.ANY),
            pl.BlockSpec((BT, D), lambda i: (i, 0)),
            pl.BlockSpec((BT, D), lambda i: (i + off, 0)),
            pl.BlockSpec((OUT, 2 * D), lambda i: (0, 0)),
            pl.BlockSpec((1, OUT), lambda i: (0, 0)),
        ],
        out_specs=pl.BlockSpec((BT, OUT), lambda i: (i + off, 0)),
        out_shape=jax.ShapeDtypeStruct((N, OUT), jnp.float32),
        input_output_aliases={0: 0},
    )(head_out, h_pool, node_feat, W, b2)


def _sc_segmax(mailbox_h, row_start, rows):
    ng = rows // G
    trips = -(-ng // NW)
    trips += trips % 2
    npairs = trips // 2

    def body(mb_hbm, out_hbm, in0, in1, out_buf, sem0, sem1, out_sem):
        wid = lax.axis_index("s") * NC + lax.axis_index("c")

        def src(g):
            return mb_hbm.at[pl.ds(row_start + g * G, G)]

        def gidx(j):
            return jnp.minimum(wid + NW * j, ng - 1)

        def compute(in_buf, j):
            def row(r, carry):
                for c in range(D // 16):
                    sl = pl.ds(c * 16, 16)
                    acc = in_buf[r, 0, sl]
                    for k in range(1, K):
                        acc = jnp.maximum(acc, in_buf[r, k, sl])
                    out_buf[j * G + r, sl] = acc
                return carry
            lax.fori_loop(0, G, row, 0)

        pltpu.make_async_copy(src(gidx(0)), in0, sem0).start()

        def pair(p, carry):
            j0 = 2 * p
            g0 = gidx(j0)
            g1 = gidx(j0 + 1)
            pltpu.make_async_copy(src(g0), in0, sem0).wait()
            pltpu.make_async_copy(src(g1), in1, sem1).start()
            compute(in0, j0)
            pltpu.make_async_copy(
                out_buf.at[pl.ds(j0 * G, G)], out_hbm.at[pl.ds(g0 * G, G)], out_sem
            ).start()
            pltpu.make_async_copy(src(g1), in1, sem1).wait()

            @pl.when(j0 + 2 < trips)
            def _():
                pltpu.make_async_copy(src(gidx(j0 + 2)), in0, sem0).start()

            compute(in1, j0 + 1)
            pltpu.make_async_copy(
                out_buf.at[pl.ds((j0 + 1) * G, G)],
                out_hbm.at[pl.ds(g1 * G, G)],
                out_sem,
            ).start()
            return carry

        lax.fori_loop(0, npairs, pair, 0)
        pltpu.make_async_copy(out_buf, out_hbm.at[pl.ds(0, trips * G)], out_sem).wait()

    kern = pl.kernel(
        body,
        out_type=jax.ShapeDtypeStruct((rows, D), jnp.float32),
        mesh=plsc.VectorSubcoreMesh(core_axis_name="c", subcore_axis_name="s"),
        scratch_types=[
            pltpu.VMEM((G, K, D), jnp.float32),
            pltpu.VMEM((G, K, D), jnp.float32),
            pltpu.VMEM((trips * G, D), jnp.float32),
            pltpu.SemaphoreType.DMA,
            pltpu.SemaphoreType.DMA,
            pltpu.SemaphoreType.DMA,
        ],
    )
    return kern(mailbox_h)


def kernel(mailbox_h, node_feat, W, b):
    b2 = b.reshape(1, OUT)
    h_pool = _sc_segmax(mailbox_h, S, N - S) if S < N else None
    out = _tc_fused(mailbox_h, node_feat, W, b2, S)
    if h_pool is not None:
        out = _tc_linear(out, h_pool, node_feat, W, b2, N - S)
    return out


# speedup vs baseline: 1.0017x; 1.0017x over previous
import jax
import jax.numpy as jnp
from jax import lax
from jax.experimental import pallas as pl
from jax.experimental.pallas import tpu as pltpu
from jax.experimental.pallas import tpu_sc as plsc

N = 10000
K = 32
D = 128
OUT = 128

S = 9200
BN = 400
BT = 400
G = 8
NW = 32
NC = 2


def _fused_body(mb_ref, nf_ref, w_ref, b_ref, out_ref):
    v = mb_ref[...]
    w = jnp.maximum(
        jnp.maximum(v[:, 0:8, :], v[:, 8:16, :]),
        jnp.maximum(v[:, 16:24, :], v[:, 24:32, :]),
    )
    acc = jnp.max(w, axis=1)
    dn = (((1,), (1,)), ((), ()))
    out = lax.dot_general(acc, w_ref[:, :D], dn, preferred_element_type=jnp.float32)
    out += lax.dot_general(nf_ref[...], w_ref[:, D:], dn, preferred_element_type=jnp.float32)
    out_ref[...] = out + b_ref[...]


def _tc_fused(mailbox_h, node_feat, W, b2, rows):
    grid = rows // BN
    return pl.pallas_call(
        _fused_body,
        grid=(grid,),
        in_specs=[
            pl.BlockSpec((BN, K, D), lambda i: (i, 0, 0)),
            pl.BlockSpec((BN, D), lambda i: (i, 0)),
            pl.BlockSpec((OUT, 2 * D), lambda i: (0, 0)),
            pl.BlockSpec((1, OUT), lambda i: (0, 0)),
        ],
        out_specs=pl.BlockSpec((BN, OUT), lambda i: (i, 0)),
        out_shape=jax.ShapeDtypeStruct((N, OUT), jnp.float32),
    )(mailbox_h, node_feat, W, b2)


def _linear_body(full_ref, h_ref, nf_ref, w_ref, b_ref, out_ref):
    del full_ref
    dn = (((1,), (1,)), ((), ()))
    out = lax.dot_general(h_ref[...], w_ref[:, :D], dn, preferred_element_type=jnp.float32)
    out += lax.dot_general(nf_ref[...], w_ref[:, D:], dn, preferred_element_type=jnp.float32)
    out_ref[...] = out + b_ref[...]


def _tc_linear(head_out, h_pool, node_feat, W, b2, rows):
    grid = rows // BT
    off = S // BT
    return pl.pallas_call(
        _linear_body,
        grid=(grid,),
        in_specs=[
            pl.BlockSpec(memory_space=pl.ANY),
            pl.BlockSpec((BT, D), lambda i: (i, 0)),
            pl.BlockSpec((BT, D), lambda i: (i + off, 0)),
            pl.BlockSpec((OUT, 2 * D), lambda i: (0, 0)),
            pl.BlockSpec((1, OUT), lambda i: (0, 0)),
        ],
        out_specs=pl.BlockSpec((BT, OUT), lambda i: (i + off, 0)),
        out_shape=jax.ShapeDtypeStruct((N, OUT), jnp.float32),
        input_output_aliases={0: 0},
    )(head_out, h_pool, node_feat, W, b2)


def _sc_segmax(mailbox_h, row_start, rows):
    ng = rows // G
    trips = -(-ng // NW)
    trips += trips % 2
    npairs = trips // 2

    def body(mb_hbm, out_hbm, in0, in1, out_buf, sem0, sem1, out_sem):
        wid = lax.axis_index("s") * NC + lax.axis_index("c")

        def src(g):
            return mb_hbm.at[pl.ds(row_start + g * G, G)]

        def gidx(j):
            return jnp.minimum(wid + NW * j, ng - 1)

        def compute(in_buf, j):
            def row(r, carry):
                for c in range(D // 16):
                    sl = pl.ds(c * 16, 16)
                    acc = in_buf[r, 0, sl]
                    for k in range(1, K):
                        acc = jnp.maximum(acc, in_buf[r, k, sl])
                    out_buf[j * G + r, sl] = acc
                return carry
            lax.fori_loop(0, G, row, 0)

        pltpu.make_async_copy(src(gidx(0)), in0, sem0).start()

        def pair(p, carry):
            j0 = 2 * p
            g0 = gidx(j0)
            g1 = gidx(j0 + 1)
            pltpu.make_async_copy(src(g0), in0, sem0).wait()
            pltpu.make_async_copy(src(g1), in1, sem1).start()
            compute(in0, j0)
            pltpu.make_async_copy(
                out_buf.at[pl.ds(j0 * G, G)], out_hbm.at[pl.ds(g0 * G, G)], out_sem
            ).start()
            pltpu.make_async_copy(src(g1), in1, sem1).wait()

            @pl.when(j0 + 2 < trips)
            def _():
                pltpu.make_async_copy(src(gidx(j0 + 2)), in0, sem0).start()

            compute(in1, j0 + 1)
            pltpu.make_async_copy(
                out_buf.at[pl.ds((j0 + 1) * G, G)],
                out_hbm.at[pl.ds(g1 * G, G)],
                out_sem,
            ).start()
            return carry

        lax.fori_loop(0, npairs, pair, 0)
        pltpu.make_async_copy(out_buf, out_hbm.at[pl.ds(0, trips * G)], out_sem).wait()

    kern = pl.kernel(
        body,
        out_type=jax.ShapeDtypeStruct((rows, D), jnp.float32),
        mesh=plsc.VectorSubcoreMesh(core_axis_name="c", subcore_axis_name="s"),
        scratch_types=[
            pltpu.VMEM((G, K, D), jnp.float32),
            pltpu.VMEM((G, K, D), jnp.float32),
            pltpu.VMEM((trips * G, D), jnp.float32),
            pltpu.SemaphoreType.DMA,
            pltpu.SemaphoreType.DMA,
            pltpu.SemaphoreType.DMA,
        ],
    )
    return kern(mailbox_h)


def kernel(mailbox_h, node_feat, W, b):
    b2 = b.reshape(1, OUT)
    h_pool = _sc_segmax(mailbox_h, S, N - S) if S < N else None
    out = _tc_fused(mailbox_h, node_feat, W, b2, S)
    if h_pool is not None:
        out = _tc_linear(out, h_pool, node_feat, W, b2, N - S)
    return out


# speedup vs baseline: 1.0415x; 1.0397x over previous
import jax
import jax.numpy as jnp
from jax import lax
from jax.experimental import pallas as pl
from jax.experimental.pallas import tpu as pltpu
from jax.experimental.pallas import tpu_sc as plsc

N = 10000
K = 32
D = 128
OUT = 128

S = 8000
BN = 400
BT = 1000
G = 8
NW = 32
NC = 2


def _fused_body(mb_ref, nf_ref, w_ref, b_ref, out_ref):
    v = mb_ref[...]
    w = jnp.maximum(
        jnp.maximum(v[:, 0:8, :], v[:, 8:16, :]),
        jnp.maximum(v[:, 16:24, :], v[:, 24:32, :]),
    )
    acc = jnp.max(w, axis=1)
    dn = (((1,), (1,)), ((), ()))
    out = lax.dot_general(acc, w_ref[:, :D], dn, preferred_element_type=jnp.float32)
    out += lax.dot_general(nf_ref[...], w_ref[:, D:], dn, preferred_element_type=jnp.float32)
    out_ref[...] = out + b_ref[...]


def _tc_fused(mailbox_h, node_feat, W, b2, rows):
    grid = rows // BN
    return pl.pallas_call(
        _fused_body,
        grid=(grid,),
        in_specs=[
            pl.BlockSpec((BN, K, D), lambda i: (i, 0, 0)),
            pl.BlockSpec((BN, D), lambda i: (i, 0)),
            pl.BlockSpec((OUT, 2 * D), lambda i: (0, 0)),
            pl.BlockSpec((1, OUT), lambda i: (0, 0)),
        ],
        out_specs=pl.BlockSpec((BN, OUT), lambda i: (i, 0)),
        out_shape=jax.ShapeDtypeStruct((N, OUT), jnp.float32),
    )(mailbox_h, node_feat, W, b2)


def _linear_body(full_ref, h_ref, nf_ref, w_ref, b_ref, out_ref):
    del full_ref
    dn = (((1,), (1,)), ((), ()))
    out = lax.dot_general(h_ref[...], w_ref[:, :D], dn, preferred_element_type=jnp.float32)
    out += lax.dot_general(nf_ref[...], w_ref[:, D:], dn, preferred_element_type=jnp.float32)
    out_ref[...] = out + b_ref[...]


def _tc_linear(head_out, h_pool, node_feat, W, b2, rows):
    grid = rows // BT
    off = S // BT
    return pl.pallas_call(
        _linear_body,
        grid=(grid,),
        in_specs=[
            pl.BlockSpec(memory_space=pl.ANY),
            pl.BlockSpec((BT, D), lambda i: (i, 0)),
            pl.BlockSpec((BT, D), lambda i: (i + off, 0)),
            pl.BlockSpec((OUT, 2 * D), lambda i: (0, 0)),
            pl.BlockSpec((1, OUT), lambda i: (0, 0)),
        ],
        out_specs=pl.BlockSpec((BT, OUT), lambda i: (i + off, 0)),
        out_shape=jax.ShapeDtypeStruct((N, OUT), jnp.float32),
        input_output_aliases={0: 0},
    )(head_out, h_pool, node_feat, W, b2)


def _sc_segmax(mailbox_h, row_start, rows):
    ng = rows // G
    trips = -(-ng // NW)
    trips += trips % 2
    npairs = trips // 2

    def body(mb_hbm, out_hbm, in0, in1, out_buf, sem0, sem1, out_sem):
        wid = lax.axis_index("s") * NC + lax.axis_index("c")

        def src(g):
            return mb_hbm.at[pl.ds(row_start + g * G, G)]

        def gidx(j):
            return jnp.minimum(wid + NW * j, ng - 1)

        def compute(in_buf, j):
            def row(r, carry):
                for c in range(D // 16):
                    sl = pl.ds(c * 16, 16)
                    acc = in_buf[r, 0, sl]
                    for k in range(1, K):
                        acc = jnp.maximum(acc, in_buf[r, k, sl])
                    out_buf[j * G + r, sl] = acc
                return carry
            lax.fori_loop(0, G, row, 0)

        pltpu.make_async_copy(src(gidx(0)), in0, sem0).start()

        def pair(p, carry):
            j0 = 2 * p
            g0 = gidx(j0)
            g1 = gidx(j0 + 1)
            pltpu.make_async_copy(src(g0), in0, sem0).wait()
            pltpu.make_async_copy(src(g1), in1, sem1).start()
            compute(in0, j0)
            pltpu.make_async_copy(
                out_buf.at[pl.ds(j0 * G, G)], out_hbm.at[pl.ds(g0 * G, G)], out_sem
            ).start()
            pltpu.make_async_copy(src(g1), in1, sem1).wait()

            @pl.when(j0 + 2 < trips)
            def _():
                pltpu.make_async_copy(src(gidx(j0 + 2)), in0, sem0).start()

            compute(in1, j0 + 1)
            pltpu.make_async_copy(
                out_buf.at[pl.ds((j0 + 1) * G, G)],
                out_hbm.at[pl.ds(g1 * G, G)],
                out_sem,
            ).start()
            return carry

        lax.fori_loop(0, npairs, pair, 0)
        pltpu.make_async_copy(out_buf, out_hbm.at[pl.ds(0, trips * G)], out_sem).wait()

    kern = pl.kernel(
        body,
        out_type=jax.ShapeDtypeStruct((rows, D), jnp.float32),
        mesh=plsc.VectorSubcoreMesh(core_axis_name="c", subcore_axis_name="s"),
        scratch_types=[
            pltpu.VMEM((G, K, D), jnp.float32),
            pltpu.VMEM((G, K, D), jnp.float32),
            pltpu.VMEM((trips * G, D), jnp.float32),
            pltpu.SemaphoreType.DMA,
            pltpu.SemaphoreType.DMA,
            pltpu.SemaphoreType.DMA,
        ],
    )
    return kern(mailbox_h)


def kernel(mailbox_h, node_feat, W, b):
    b2 = b.reshape(1, OUT)
    h_pool = _sc_segmax(mailbox_h, S, N - S) if S < N else None
    out = _tc_fused(mailbox_h, node_feat, W, b2, S)
    if h_pool is not None:
        out = _tc_linear(out, h_pool, node_feat, W, b2, N - S)
    return out


# speedup vs baseline: 1.0585x; 1.0163x over previous
import jax
import jax.numpy as jnp
from jax import lax
from jax.experimental import pallas as pl
from jax.experimental.pallas import tpu as pltpu
from jax.experimental.pallas import tpu_sc as plsc

N = 10000
K = 32
D = 128
OUT = 128

S = 8000
BN = 400
BT = 1000
G = 8
NC = 1
NW = 16 * NC


def _fused_body(mb_ref, nf_ref, w_ref, b_ref, out_ref):
    v = mb_ref[...]
    w = jnp.maximum(
        jnp.maximum(v[:, 0:8, :], v[:, 8:16, :]),
        jnp.maximum(v[:, 16:24, :], v[:, 24:32, :]),
    )
    acc = jnp.max(w, axis=1)
    dn = (((1,), (1,)), ((), ()))
    out = lax.dot_general(acc, w_ref[:, :D], dn, preferred_element_type=jnp.float32)
    out += lax.dot_general(nf_ref[...], w_ref[:, D:], dn, preferred_element_type=jnp.float32)
    out_ref[...] = out + b_ref[...]


def _tc_fused(mailbox_h, node_feat, W, b2, rows):
    grid = rows // BN
    return pl.pallas_call(
        _fused_body,
        grid=(grid,),
        in_specs=[
            pl.BlockSpec((BN, K, D), lambda i: (i, 0, 0)),
            pl.BlockSpec((BN, D), lambda i: (i, 0)),
            pl.BlockSpec((OUT, 2 * D), lambda i: (0, 0)),
            pl.BlockSpec((1, OUT), lambda i: (0, 0)),
        ],
        out_specs=pl.BlockSpec((BN, OUT), lambda i: (i, 0)),
        out_shape=jax.ShapeDtypeStruct((N, OUT), jnp.float32),
    )(mailbox_h, node_feat, W, b2)


def _linear_body(full_ref, h_ref, nf_ref, w_ref, b_ref, out_ref):
    del full_ref
    dn = (((1,), (1,)), ((), ()))
    out = lax.dot_general(h_ref[...], w_ref[:, :D], dn, preferred_element_type=jnp.float32)
    out += lax.dot_general(nf_ref[...], w_ref[:, D:], dn, preferred_element_type=jnp.float32)
    out_ref[...] = out + b_ref[...]


def _tc_linear(head_out, h_pool, node_feat, W, b2, rows):
    grid = rows // BT
    off = S // BT
    return pl.pallas_call(
        _linear_body,
        grid=(grid,),
        in_specs=[
            pl.BlockSpec(memory_space=pl.ANY),
            pl.BlockSpec((BT, D), lambda i: (i, 0)),
            pl.BlockSpec((BT, D), lambda i: (i + off, 0)),
            pl.BlockSpec((OUT, 2 * D), lambda i: (0, 0)),
            pl.BlockSpec((1, OUT), lambda i: (0, 0)),
        ],
        out_specs=pl.BlockSpec((BT, OUT), lambda i: (i + off, 0)),
        out_shape=jax.ShapeDtypeStruct((N, OUT), jnp.float32),
        input_output_aliases={0: 0},
    )(head_out, h_pool, node_feat, W, b2)


def _sc_segmax(mailbox_h, row_start, rows):
    ng = rows // G
    trips = -(-ng // NW)
    trips += trips % 2
    npairs = trips // 2

    def body(mb_hbm, out_hbm, in0, in1, out_buf, sem0, sem1, out_sem):
        wid = lax.axis_index("s") * NC + lax.axis_index("c")

        def src(g):
            return mb_hbm.at[pl.ds(row_start + g * G, G)]

        def gidx(j):
            return jnp.minimum(wid + NW * j, ng - 1)

        def compute(in_buf, j):
            def row(r, carry):
                for c in range(D // 16):
                    sl = pl.ds(c * 16, 16)
                    acc = in_buf[r, 0, sl]
                    for k in range(1, K):
                        acc = jnp.maximum(acc, in_buf[r, k, sl])
                    out_buf[j * G + r, sl] = acc
                return carry
            lax.fori_loop(0, G, row, 0)

        pltpu.make_async_copy(src(gidx(0)), in0, sem0).start()

        def pair(p, carry):
            j0 = 2 * p
            g0 = gidx(j0)
            g1 = gidx(j0 + 1)
            pltpu.make_async_copy(src(g0), in0, sem0).wait()
            pltpu.make_async_copy(src(g1), in1, sem1).start()
            compute(in0, j0)
            pltpu.make_async_copy(
                out_buf.at[pl.ds(j0 * G, G)], out_hbm.at[pl.ds(g0 * G, G)], out_sem
            ).start()
            pltpu.make_async_copy(src(g1), in1, sem1).wait()

            @pl.when(j0 + 2 < trips)
            def _():
                pltpu.make_async_copy(src(gidx(j0 + 2)), in0, sem0).start()

            compute(in1, j0 + 1)
            pltpu.make_async_copy(
                out_buf.at[pl.ds((j0 + 1) * G, G)],
                out_hbm.at[pl.ds(g1 * G, G)],
                out_sem,
            ).start()
            return carry

        lax.fori_loop(0, npairs, pair, 0)
        pltpu.make_async_copy(out_buf, out_hbm.at[pl.ds(0, trips * G)], out_sem).wait()

    kern = pl.kernel(
        body,
        out_type=jax.ShapeDtypeStruct((rows, D), jnp.float32),
        mesh=plsc.VectorSubcoreMesh(core_axis_name="c", subcore_axis_name="s", num_cores=NC),
        scratch_types=[
            pltpu.VMEM((G, K, D), jnp.float32),
            pltpu.VMEM((G, K, D), jnp.float32),
            pltpu.VMEM((trips * G, D), jnp.float32),
            pltpu.SemaphoreType.DMA,
            pltpu.SemaphoreType.DMA,
            pltpu.SemaphoreType.DMA,
        ],
    )
    return kern(mailbox_h)


def kernel(mailbox_h, node_feat, W, b):
    b2 = b.reshape(1, OUT)
    h_pool = _sc_segmax(mailbox_h, S, N - S) if S < N else None
    out = _tc_fused(mailbox_h, node_feat, W, b2, S)
    if h_pool is not None:
        out = _tc_linear(out, h_pool, node_feat, W, b2, N - S)
    return out
